# baseline (speedup 1.0000x reference)
import jax
import jax.numpy as jnp
from jax import lax
from jax.experimental import pallas as pl
from jax.experimental.pallas import tpu as pltpu
from jax.experimental.pallas import tpu_sc as plsc

VOCAB = 1000000
EMB_DIM = 64
BATCH = 16384

_NC = 2
_NS = 16
_LANES = 16
_NW = _NC * _NS
_BPW = BATCH // _NW
_ICHUNK = 128
_NCHUNK = _BPW // _ICHUNK


def _emb_mul_kernel(idx_hbm, repr_hbm, table_hbm, out_hbm,
                    idx_v, rows_v, rep_v, gsem, rsem):
    wid = lax.axis_index("s") * _NC + lax.axis_index("c")
    base = wid * _BPW

    pltpu.sync_copy(idx_hbm.at[wid], idx_v)

    rep_cp = pltpu.async_copy(repr_hbm.at[pl.ds(base, _BPW)], rep_v, rsem)

    gathers = []
    for j in range(_NCHUNK):
        gathers.append(pltpu.async_copy(
            table_hbm.at[idx_v.at[j]],
            rows_v.at[pl.ds(j * _ICHUNK, _ICHUNK)],
            gsem))
    for cp in gathers:
        cp.wait()
    rep_cp.wait()

    def body(i, carry):
        for c in range(EMB_DIM // _LANES):
            sl = pl.ds(c * _LANES, _LANES)
            rows_v[i, sl] = rows_v[i, sl] * rep_v[i, sl]
        return carry

    lax.fori_loop(0, _BPW, body, 0, unroll=4)

    pltpu.sync_copy(rows_v, out_hbm.at[pl.ds(base, _BPW)])


@jax.jit
def kernel(_next_types, representation, table):
    idx = _next_types.reshape(_NW, _NCHUNK, _ICHUNK).astype(jnp.int32)
    rep = representation.reshape(BATCH, EMB_DIM)

    mesh = plsc.VectorSubcoreMesh(core_axis_name="c", subcore_axis_name="s")
    out = pl.kernel(
        _emb_mul_kernel,
        out_type=jax.ShapeDtypeStruct((BATCH, EMB_DIM), jnp.float32),
        mesh=mesh,
        compiler_params=pltpu.CompilerParams(use_tc_tiling_on_sc=False),
        scratch_types=[
            pltpu.VMEM((_NCHUNK, _ICHUNK), jnp.int32),
            pltpu.VMEM((_BPW, EMB_DIM), jnp.float32),
            pltpu.VMEM((_BPW, EMB_DIM), jnp.float32),
            pltpu.SemaphoreType.DMA,
            pltpu.SemaphoreType.DMA,
        ],
    )(idx, rep, table)
    return out.reshape(BATCH, 1, EMB_DIM)
